# baseline (speedup 1.0000x reference)
import jax
import jax.numpy as jnp
from jax.experimental import pallas as pl
from jax.experimental.pallas import tpu as pltpu

_NSIGMA = 3.0
_PATCH = 8
_BN = 1000
_TINY = 1e-30


def _raster_body(c_ref, s_ref, ch_ref, h_ref, out_ref, off_ref):
    c = c_ref[0]
    s = s_ref[0]
    inv_sqrt2 = 0.7071067811865476

    ir3 = jax.lax.broadcasted_iota(jnp.int32, (3, 1), 0)
    h = jnp.where(ir3 == 0, h_ref[0], jnp.where(ir3 == 1, h_ref[1], h_ref[2]))

    offf = jnp.floor((c - _NSIGMA * s) / h)
    invs = inv_sqrt2 / s
    b0 = (offf * h - c) * invs
    step = h * invs

    ch = ch_ref[0]
    cdf_prev = 0.5 * (1.0 + jax.lax.erf(b0))
    q0r, q1r, q2r = [], [], []
    for t in range(1, _PATCH + 1):
        cdf = 0.5 * (1.0 + jax.lax.erf(b0 + float(t) * step))
        d = cdf - cdf_prev
        q0r.append(d[0:1] * ch)
        q1r.append(d[1:2])
        q2r.append(d[2:3])
        cdf_prev = cdf
    q0c = jnp.concatenate(q0r, axis=0)
    q1 = jnp.concatenate(q1r, axis=0)
    q2 = jnp.concatenate(q2r, axis=0)

    def dot_t(lhs, rhs):
        return jax.lax.dot_general(lhs, rhs, (((0,), (0,)), ((), ())),
                                   preferred_element_type=jnp.float32)

    ip = jax.lax.broadcasted_iota(jnp.int32, (8, 64), 1)
    ir8 = jax.lax.broadcasted_iota(jnp.int32, (8, 64), 0)
    e0 = (ip // 8 == ir8).astype(jnp.float32)
    e1 = (ip % 8 == ir8).astype(jnp.float32)
    im = jax.lax.broadcasted_iota(jnp.int32, (8, 512), 1)
    ir5 = jax.lax.broadcasted_iota(jnp.int32, (8, 512), 0)
    e2 = (im % 8 == ir5).astype(jnp.float32)
    ig = jax.lax.broadcasted_iota(jnp.int32, (64, 512), 1)
    irg = jax.lax.broadcasted_iota(jnp.int32, (64, 512), 0)
    g = (ig // 8 == irg).astype(jnp.float32)

    t01 = dot_t(q0c, e0) * dot_t(q1, e1)
    out_ref[...] = jnp.dot(t01, g, preferred_element_type=jnp.float32) * dot_t(q2, e2)
    off_ref[0] = offf.astype(jnp.int32)


def kernel(sigma, time, charge, tail, grid_spacing, velocity):
    n = sigma.shape[0]
    grid = n // _BN
    c_t = jnp.stack([tail[:, 1], tail[:, 0], time]).reshape(3, grid, _BN)
    c_t = c_t.transpose(1, 0, 2)
    s_t = sigma.T.reshape(3, grid, _BN).transpose(1, 0, 2)
    ch_t = charge.reshape(grid, 1, _BN)
    rasters, offsets_t = pl.pallas_call(
        _raster_body,
        grid=(grid,),
        in_specs=[
            pl.BlockSpec((1, 3, _BN), lambda i: (i, 0, 0)),
            pl.BlockSpec((1, 3, _BN), lambda i: (i, 0, 0)),
            pl.BlockSpec((1, 1, _BN), lambda i: (i, 0, 0)),
            pl.BlockSpec(memory_space=pltpu.SMEM),
        ],
        out_specs=[
            pl.BlockSpec((_BN, 512), lambda i: (i, 0)),
            pl.BlockSpec((1, 3, _BN), lambda i: (i, 0, 0)),
        ],
        out_shape=[
            jax.ShapeDtypeStruct((n, 512), jnp.float32),
            jax.ShapeDtypeStruct((grid, 3, _BN), jnp.int32),
        ],
    )(c_t, s_t, ch_t, grid_spacing)
    offsets = offsets_t.transpose(1, 0, 2).reshape(3, n).T
    return rasters.reshape(n, _PATCH, _PATCH, _PATCH), offsets


# speedup vs baseline: 1.0875x; 1.0875x over previous
import jax
import jax.numpy as jnp
from jax.experimental import pallas as pl
from jax.experimental.pallas import tpu as pltpu

_NSIGMA = 3.0
_PATCH = 8
_BN = 2000
_TINY = 1e-30


def _raster_body(c_ref, s_ref, ch_ref, h_ref, out_ref, off_ref):
    c = c_ref[0]
    s = s_ref[0]
    inv_sqrt2 = 0.7071067811865476

    ir3 = jax.lax.broadcasted_iota(jnp.int32, (3, 1), 0)
    h = jnp.where(ir3 == 0, h_ref[0], jnp.where(ir3 == 1, h_ref[1], h_ref[2]))

    offf = jnp.floor((c - _NSIGMA * s) / h)
    invs = inv_sqrt2 / s
    b0 = (offf * h - c) * invs
    step = h * invs

    ch = ch_ref[0]
    cdf_prev = 0.5 * (1.0 + jax.lax.erf(b0))
    q0r, q1r, q2r = [], [], []
    for t in range(1, _PATCH + 1):
        cdf = 0.5 * (1.0 + jax.lax.erf(b0 + float(t) * step))
        d = cdf - cdf_prev
        q0r.append(d[0:1] * ch)
        q1r.append(d[1:2])
        q2r.append(d[2:3])
        cdf_prev = cdf
    q0c = jnp.concatenate(q0r, axis=0)
    q1 = jnp.concatenate(q1r, axis=0)
    q2 = jnp.concatenate(q2r, axis=0)

    def dot_t(lhs, rhs):
        return jax.lax.dot_general(lhs, rhs, (((0,), (0,)), ((), ())),
                                   preferred_element_type=jnp.float32)

    ip = jax.lax.broadcasted_iota(jnp.int32, (8, 64), 1)
    ir8 = jax.lax.broadcasted_iota(jnp.int32, (8, 64), 0)
    e0 = (ip // 8 == ir8).astype(jnp.float32)
    e1 = (ip % 8 == ir8).astype(jnp.float32)
    im = jax.lax.broadcasted_iota(jnp.int32, (8, 512), 1)
    ir5 = jax.lax.broadcasted_iota(jnp.int32, (8, 512), 0)
    e2 = (im % 8 == ir5).astype(jnp.float32)
    ig = jax.lax.broadcasted_iota(jnp.int32, (64, 512), 1)
    irg = jax.lax.broadcasted_iota(jnp.int32, (64, 512), 0)
    g = (ig // 8 == irg).astype(jnp.float32)

    t01 = dot_t(q0c, e0) * dot_t(q1, e1)
    out_ref[...] = jnp.dot(t01, g, preferred_element_type=jnp.float32) * dot_t(q2, e2)
    off_ref[0] = offf.astype(jnp.int32)


def kernel(sigma, time, charge, tail, grid_spacing, velocity):
    n = sigma.shape[0]
    grid = n // _BN
    c_t = jnp.stack([tail[:, 1], tail[:, 0], time]).reshape(3, grid, _BN)
    c_t = c_t.transpose(1, 0, 2)
    s_t = sigma.T.reshape(3, grid, _BN).transpose(1, 0, 2)
    ch_t = charge.reshape(grid, 1, _BN)
    rasters, offsets_t = pl.pallas_call(
        _raster_body,
        grid=(grid,),
        in_specs=[
            pl.BlockSpec((1, 3, _BN), lambda i: (i, 0, 0)),
            pl.BlockSpec((1, 3, _BN), lambda i: (i, 0, 0)),
            pl.BlockSpec((1, 1, _BN), lambda i: (i, 0, 0)),
            pl.BlockSpec(memory_space=pltpu.SMEM),
        ],
        out_specs=[
            pl.BlockSpec((_BN, 512), lambda i: (i, 0)),
            pl.BlockSpec((1, 3, _BN), lambda i: (i, 0, 0)),
        ],
        out_shape=[
            jax.ShapeDtypeStruct((n, 512), jnp.float32),
            jax.ShapeDtypeStruct((grid, 3, _BN), jnp.int32),
        ],
    )(c_t, s_t, ch_t, grid_spacing)
    offsets = offsets_t.transpose(1, 0, 2).reshape(3, n).T
    return rasters.reshape(n, _PATCH, _PATCH, _PATCH), offsets
